# Initial kernel scaffold; baseline (speedup 1.0000x reference)
import jax
import jax.numpy as jnp
from jax.experimental import pallas as pl


def kernel(x):
    raise NotImplementedError("write your pallas kernel here")


# speedup vs baseline: 24.2001x; 24.2001x over previous
import jax
import jax.numpy as jnp
from jax.experimental import pallas as pl

GAMMA_K = 0.1
_INT_MIN = -(2 ** 31)


def _select_mask_body(x_ref, o_ref, *, kth: int):
    xb = x_ref[0]
    xz = xb + 0.0
    b = jax.lax.bitcast_convert_type(xz, jnp.int32)
    u = jnp.where(b < 0, b ^ jnp.int32(0x7FFFFFFF), b)
    imin = jnp.int32(_INT_MIN)

    def it(i, zb):
        bit = jnp.int32(31) - i
        cand = zb | jnp.left_shift(jnp.int32(1), bit)
        z = cand ^ imin
        cnt = jnp.sum((u >= z).astype(jnp.int32))
        return jnp.where(cnt >= kth, cand, zb)

    zb = jax.lax.fori_loop(0, 32, it, jnp.int32(0))
    t = zb ^ imin
    o_ref[0] = jnp.where(u < t, xb, 0.0)


def kernel(x):
    B, C, H, W = x.shape
    n = C * H * W
    kth = int(GAMMA_K * n)
    lanes = 1024
    rows = n // lanes
    xf = x.reshape(B, rows, lanes)

    out = pl.pallas_call(
        lambda x_ref, o_ref: _select_mask_body(x_ref, o_ref, kth=kth),
        grid=(B,),
        in_specs=[pl.BlockSpec((1, rows, lanes), lambda i: (i, 0, 0))],
        out_specs=pl.BlockSpec((1, rows, lanes), lambda i: (i, 0, 0)),
        out_shape=jax.ShapeDtypeStruct((B, rows, lanes), jnp.float32),
    )(xf)
    return out.reshape(B, C, H, W)
